# baseline (speedup 1.0000x reference)
import functools

import jax
import jax.numpy as jnp
from jax import lax
from jax.experimental import pallas as pl
from jax.experimental.pallas import tpu as pltpu
from jax.experimental.pallas import tpu_sc as plsc

D_MODEL = 32
NUM_CORES = 2
NUM_SUBCORES = 16
NW = NUM_CORES * NUM_SUBCORES
GROUP = 128
K = 13

NROW_PAD = 1000064
NPANEL = NROW_PAD // 128


@jax.jit
def _relayout(tT, tail):
    mesh = plsc.VectorSubcoreMesh(core_axis_name="c", subcore_axis_name="s")
    base, extra = divmod(NPANEL - 1, NW)

    @functools.partial(
        pl.kernel,
        out_type=jax.ShapeDtypeStruct((NROW_PAD * D_MODEL // 128, 128),
                                      jnp.float32),
        mesh=mesh,
        scratch_types=[
            pltpu.VMEM((2, D_MODEL, 128), jnp.float32),
            pltpu.VMEM((2, D_MODEL, 128), jnp.float32),
            pltpu.SemaphoreType.DMA,
            pltpu.SemaphoreType.DMA,
        ],
        compiler_params=pltpu.CompilerParams(needs_layout_passes=False),
    )
    def k(t_hbm, tail_hbm, out_hbm, tin, tout, isem, osem):
        wid = lax.axis_index("s") * NUM_CORES + lax.axis_index("c")
        jstart = wid * base + lax.min(wid, extra)
        cnt = base + (wid < extra).astype(jnp.int32)
        PANEL_W = D_MODEL * 128

        @pl.when(wid == NW - 1)
        def _():
            pltpu.sync_copy(tail_hbm,
                            out_hbm.at[pl.ds((NROW_PAD - 128) // 4, 16)])

        def panel_src(j):
            return t_hbm.at[:, pl.ds(pl.multiple_of(j * 128, 128), 128)]

        pltpu.async_copy(panel_src(jstart), tin.at[0], isem)

        iota = lax.broadcasted_iota(jnp.int32, (16,), 0)
        bbk = [jnp.bitwise_and(iota + kd, 15) for kd in range(16)]
        wk = [jnp.bitwise_and(iota + kd, 15) * D_MODEL + iota
              for kd in range(16)]
        crow = [iota + 16 * h for h in range(D_MODEL // 16)]

        def step(kk, carry):
            b = lax.rem(kk, 2)
            j = jstart + kk
            pltpu.make_async_copy(panel_src(j), tin.at[b], isem).wait()

            @pl.when(kk + 1 < cnt)
            def _():
                pltpu.async_copy(panel_src(j + 1), tin.at[1 - b], isem)

            @pl.when(kk >= 2)
            def _():
                pltpu.make_async_copy(
                    tout.at[0], out_hbm.at[pl.ds(0, D_MODEL)], osem
                ).wait()

            src = tin.at[b]
            dst = tout.at[b]
            for h in range(D_MODEL // 16):
                for bb0 in range(0, 128, 16):
                    batch = []
                    for kd in range(16):
                        v = plsc.load_gather(src, [crow[h], bbk[kd] + bb0])
                        batch.append((wk[kd] + (bb0 * D_MODEL + 16 * h), v))
                    for widx, v in batch:
                        plsc.store_scatter(
                            dst,
                            [lax.shift_right_logical(widx, 7),
                             jnp.bitwise_and(widx, 127)],
                            v,
                        )

            pltpu.async_copy(
                dst,
                out_hbm.at[pl.ds(pl.multiple_of(j * D_MODEL, 8), D_MODEL)],
                osem,
            )
            return carry

        lax.fori_loop(0, cnt, step, 0)
        for _ in range(2):
            pltpu.make_async_copy(
                tout.at[0], out_hbm.at[pl.ds(0, D_MODEL)], osem
            ).wait()

    return k(tT, tail)


@jax.jit
def _gather_native(table_lin, xTp):
    NB = 16384 // 128
    NF = 26
    JB_PER_W = NB // NW
    NITEM = JB_PER_W * 4
    mesh = plsc.VectorSubcoreMesh(core_axis_name="c", subcore_axis_name="s")

    @functools.partial(
        pl.kernel,
        out_type=jax.ShapeDtypeStruct((NF, 4, NB, 8, 128), jnp.float32),
        mesh=mesh,
        scratch_types=[
            pltpu.VMEM((2, 8, 128), jnp.int32),
            pltpu.VMEM((2, 1024, D_MODEL), jnp.float32),
            pltpu.VMEM((8, D_MODEL, 128), jnp.float32),
            pltpu.SemaphoreType.DMA,
            pltpu.SemaphoreType.DMA,
            pltpu.SemaphoreType.DMA,
        ],
        compiler_params=pltpu.CompilerParams(
            use_tc_tiling_on_sc=False, needs_layout_passes=False
        ),
    )
    def k(table_hbm, x_hbm, out_hbm, idx_v, rows_v, tile, gsem0, gsem1, ssem):
        wid = lax.axis_index("s") * NUM_CORES + lax.axis_index("c")

        iota = lax.broadcasted_iota(jnp.int32, (16,), 0)

        def item_jb_g0(n):
            return wid * JB_PER_W + n // 4, lax.rem(n, 4) * 8

        def load_idx(n, buf):
            jb, g0 = item_jb_g0(n)
            pltpu.sync_copy(
                x_hbm.at[pl.ds(g0, 8), pl.ds(jb * 128, 128)], idx_v.at[buf]
            )

        def fire_gathers(buf, sem):
            for u in range(8):
                pltpu.async_copy(
                    table_hbm.at[idx_v.at[buf, u]],
                    rows_v.at[buf, pl.ds(u * 128, 128)],
                    sem,
                )

        load_idx(0, 0)
        fire_gathers(0, gsem0)

        def drain_store():
            pltpu.make_async_copy(
                tile.at[0, pl.ds(0, 8)], out_hbm.at[0, 0, 0], ssem
            ).wait()

        def step(t, carry):
            n = t // 8
            u = lax.rem(t, 8)
            buf = lax.rem(n, 2)
            jb, g0 = item_jb_g0(n)
            f = g0 + u

            @pl.when(u == 0)
            def _():
                @pl.when(n + 1 < NITEM)
                def _():
                    load_idx(n + 1, 1 - buf)

                    @pl.when(lax.rem(n, 2) == 0)
                    def _():
                        fire_gathers(1 - buf, gsem1)

                    @pl.when(lax.rem(n, 2) == 1)
                    def _():
                        fire_gathers(1 - buf, gsem0)

                @pl.when(lax.rem(n, 2) == 0)
                def _():
                    pltpu.make_async_copy(
                        table_hbm.at[pl.ds(0, 1024)], rows_v.at[buf], gsem0
                    ).wait()

                @pl.when(lax.rem(n, 2) == 1)
                def _():
                    pltpu.make_async_copy(
                        table_hbm.at[pl.ds(0, 1024)], rows_v.at[buf], gsem1
                    ).wait()

            src = rows_v.at[buf]
            dst = tile.at[u]

            def tblock(bo, carry3):
                b0 = bo * 16
                for h in range(D_MODEL // 16):
                    crow = iota + 16 * h
                    batch = []
                    for kd in range(16):
                        skew = jnp.bitwise_and(iota + kd, 15)
                        v = plsc.load_gather(
                            src, [skew + (u * 128 + b0), crow]
                        )
                        batch.append((skew, v))
                    for skew, v in batch:
                        plsc.store_scatter(dst, [crow, skew + b0], v)
                return carry3

            lax.fori_loop(0, 8, tblock, 0)

            @pl.when(f < NF)
            def _():
                for i in range(4):
                    pltpu.async_copy(
                        dst.at[pl.ds(8 * i, 8)],
                        out_hbm.at[f, i, jb],
                        ssem,
                    )

            @pl.when(u == 7)
            def _():
                for _ in range(8):
                    drain_store()

                @pl.when(g0 < 24)
                def _():
                    for _ in range(24):
                        drain_store()

            return carry

        lax.fori_loop(0, NITEM * 8, step, 0)

    return k(table_lin, xTp)


@functools.partial(jax.jit, static_argnames=("nchunk",))
def _gather_rows(idx, table_lin, nchunk):
    G = nchunk * K
    mesh = plsc.VectorSubcoreMesh(core_axis_name="c", subcore_axis_name="s")

    @functools.partial(
        pl.kernel,
        out_type=jax.ShapeDtypeStruct((NW, G, GROUP, D_MODEL), jnp.float32),
        mesh=mesh,
        scratch_types=[
            pltpu.VMEM((G, GROUP), jnp.int32),
            pltpu.VMEM((2, K, GROUP, D_MODEL), jnp.float32),
            pltpu.SemaphoreType.DMA,
            pltpu.SemaphoreType.DMA,
            pltpu.SemaphoreType.DMA,
        ],
        compiler_params=pltpu.CompilerParams(use_tc_tiling_on_sc=False),
    )
    def k(table_hbm, idx_hbm, out_hbm, idx_v, rows_v, gsem0, gsem1, ssem):
        wid = lax.axis_index("s") * NUM_CORES + lax.axis_index("c")
        pltpu.sync_copy(idx_hbm.at[wid], idx_v)

        def fire(c, region, sem):
            for j in range(K):
                pltpu.async_copy(
                    table_hbm.at[idx_v.at[c * K + j]],
                    rows_v.at[region, j],
                    sem,
                )

        fire(0, 0, gsem0)

        def step(c, carry):
            r = lax.rem(c, 2)
            cur_sem_is0 = lax.rem(c, 2) == 0

            @pl.when(c >= 1)
            def _():
                pltpu.make_async_copy(
                    rows_v.at[0], out_hbm.at[wid, pl.ds(0, K)], ssem
                ).wait()

            @pl.when(c + 1 < nchunk)
            def _():
                @pl.when(cur_sem_is0)
                def _():
                    fire(c + 1, 1 - r, gsem1)

                @pl.when(jnp.logical_not(cur_sem_is0))
                def _():
                    fire(c + 1, 1 - r, gsem0)

            @pl.when(cur_sem_is0)
            def _():
                pltpu.make_async_copy(
                    out_hbm.at[wid, pl.ds(0, K)], rows_v.at[r], gsem0
                ).wait()

            @pl.when(jnp.logical_not(cur_sem_is0))
            def _():
                pltpu.make_async_copy(
                    out_hbm.at[wid, pl.ds(0, K)], rows_v.at[r], gsem1
                ).wait()

            pltpu.async_copy(rows_v.at[r], out_hbm.at[wid, pl.ds(c * K, K)],
                             ssem)
            return carry

        lax.fori_loop(0, nchunk, step, 0)
        pltpu.make_async_copy(
            rows_v.at[0], out_hbm.at[wid, pl.ds(0, K)], ssem
        ).wait()

    return k(table_lin, idx)


def kernel(x, table):
    tail = table[NROW_PAD - 128:].reshape(16, 128)
    table_lin = _relayout(table.T, tail).reshape(NROW_PAD, D_MODEL)
    xTp = jnp.pad(x.T, ((0, 32 - x.shape[1]), (0, 0)))
    out5 = _gather_native(table_lin, xTp)
    return out5.transpose(2, 4, 0, 1, 3).reshape(
        x.shape[0], x.shape[1], D_MODEL)


# speedup vs baseline: 2.2839x; 2.2839x over previous
import functools

import jax
import jax.numpy as jnp
from jax import lax
from jax.experimental import pallas as pl
from jax.experimental.pallas import tpu as pltpu
from jax.experimental.pallas import tpu_sc as plsc

D_MODEL = 32
NUM_CORES = 2
NUM_SUBCORES = 16
NW = NUM_CORES * NUM_SUBCORES
GROUP = 128
K = 13

NROW_PAD = 1000064
NPANEL = NROW_PAD // 128


@jax.jit
def _relayout(tT, tail):
    mesh = plsc.VectorSubcoreMesh(core_axis_name="c", subcore_axis_name="s")
    base, extra = divmod(NPANEL - 1, NW)

    @functools.partial(
        pl.kernel,
        out_type=jax.ShapeDtypeStruct((NROW_PAD * D_MODEL // 128, 128),
                                      jnp.float32),
        mesh=mesh,
        scratch_types=[
            pltpu.VMEM((2, D_MODEL, 128), jnp.float32),
            pltpu.VMEM((2, D_MODEL, 128), jnp.float32),
            pltpu.SemaphoreType.DMA,
            pltpu.SemaphoreType.DMA,
        ],
        compiler_params=pltpu.CompilerParams(needs_layout_passes=False),
    )
    def k(t_hbm, tail_hbm, out_hbm, tin, tout, isem, osem):
        wid = lax.axis_index("s") * NUM_CORES + lax.axis_index("c")
        jstart = wid * base + lax.min(wid, extra)
        cnt = base + (wid < extra).astype(jnp.int32)
        PANEL_W = D_MODEL * 128

        @pl.when(wid == NW - 1)
        def _():
            pltpu.sync_copy(tail_hbm,
                            out_hbm.at[pl.ds((NROW_PAD - 128) // 4, 16)])

        def panel_src(j):
            return t_hbm.at[:, pl.ds(pl.multiple_of(j * 128, 128), 128)]

        pltpu.async_copy(panel_src(jstart), tin.at[0], isem)

        iota = lax.broadcasted_iota(jnp.int32, (16,), 0)
        bbk = [jnp.bitwise_and(iota + kd, 15) for kd in range(16)]
        wk = [jnp.bitwise_and(iota + kd, 15) * D_MODEL + iota
              for kd in range(16)]
        crow = [iota + 16 * h for h in range(D_MODEL // 16)]

        def step(kk, carry):
            b = lax.rem(kk, 2)
            j = jstart + kk
            pltpu.make_async_copy(panel_src(j), tin.at[b], isem).wait()

            @pl.when(kk + 1 < cnt)
            def _():
                pltpu.async_copy(panel_src(j + 1), tin.at[1 - b], isem)

            @pl.when(kk >= 2)
            def _():
                pltpu.make_async_copy(
                    tout.at[0], out_hbm.at[pl.ds(0, D_MODEL)], osem
                ).wait()

            src = tin.at[b]
            dst = tout.at[b]
            for h in range(D_MODEL // 16):
                for bb0 in range(0, 128, 16):
                    batch = []
                    for kd in range(16):
                        v = plsc.load_gather(src, [crow[h], bbk[kd] + bb0])
                        batch.append((wk[kd] + (bb0 * D_MODEL + 16 * h), v))
                    for widx, v in batch:
                        plsc.store_scatter(
                            dst,
                            [lax.shift_right_logical(widx, 7),
                             jnp.bitwise_and(widx, 127)],
                            v,
                        )

            pltpu.async_copy(
                dst,
                out_hbm.at[pl.ds(pl.multiple_of(j * D_MODEL, 8), D_MODEL)],
                osem,
            )
            return carry

        lax.fori_loop(0, cnt, step, 0)
        for _ in range(2):
            pltpu.make_async_copy(
                tout.at[0], out_hbm.at[pl.ds(0, D_MODEL)], osem
            ).wait()

    return k(tT, tail)


@functools.partial(jax.jit, static_argnames=("nchunk",))
def _gather_rows(idx, table_lin, nchunk):
    G = nchunk * K
    mesh = plsc.VectorSubcoreMesh(core_axis_name="c", subcore_axis_name="s")

    @functools.partial(
        pl.kernel,
        out_type=jax.ShapeDtypeStruct((NW, G, GROUP, D_MODEL), jnp.float32),
        mesh=mesh,
        scratch_types=[
            pltpu.VMEM((G, GROUP), jnp.int32),
            pltpu.VMEM((2, K, GROUP, D_MODEL), jnp.float32),
            pltpu.SemaphoreType.DMA,
            pltpu.SemaphoreType.DMA,
            pltpu.SemaphoreType.DMA,
        ],
        compiler_params=pltpu.CompilerParams(use_tc_tiling_on_sc=False),
    )
    def k(table_hbm, idx_hbm, out_hbm, idx_v, rows_v, gsem0, gsem1, ssem):
        wid = lax.axis_index("s") * NUM_CORES + lax.axis_index("c")
        pltpu.sync_copy(idx_hbm.at[wid], idx_v)

        def fire(c, region, sem):
            for j in range(K):
                pltpu.async_copy(
                    table_hbm.at[idx_v.at[c * K + j]],
                    rows_v.at[region, j],
                    sem,
                )

        fire(0, 0, gsem0)

        def step(c, carry):
            r = lax.rem(c, 2)
            cur_sem_is0 = lax.rem(c, 2) == 0

            @pl.when(c >= 1)
            def _():
                pltpu.make_async_copy(
                    rows_v.at[0], out_hbm.at[wid, pl.ds(0, K)], ssem
                ).wait()

            @pl.when(c + 1 < nchunk)
            def _():
                @pl.when(cur_sem_is0)
                def _():
                    fire(c + 1, 1 - r, gsem1)

                @pl.when(jnp.logical_not(cur_sem_is0))
                def _():
                    fire(c + 1, 1 - r, gsem0)

            @pl.when(cur_sem_is0)
            def _():
                pltpu.make_async_copy(
                    out_hbm.at[wid, pl.ds(0, K)], rows_v.at[r], gsem0
                ).wait()

            @pl.when(jnp.logical_not(cur_sem_is0))
            def _():
                pltpu.make_async_copy(
                    out_hbm.at[wid, pl.ds(0, K)], rows_v.at[r], gsem1
                ).wait()

            pltpu.async_copy(rows_v.at[r], out_hbm.at[wid, pl.ds(c * K, K)],
                             ssem)
            return carry

        lax.fori_loop(0, nchunk, step, 0)
        pltpu.make_async_copy(
            rows_v.at[0], out_hbm.at[wid, pl.ds(0, K)], ssem
        ).wait()

    return k(table_lin, idx)


def kernel(x, table):
    B_total = x.shape[0] * x.shape[1]
    chunk = NW * GROUP * K
    B_pad = ((B_total + chunk - 1) // chunk) * chunk
    nchunk = B_pad // chunk
    G = nchunk * K
    xf = x.reshape(-1)
    if B_pad != B_total:
        xf = jnp.concatenate(
            [xf, jnp.zeros((B_pad - B_total,), dtype=xf.dtype)]
        )
    idx = xf.reshape(NW, G, GROUP)
    tail = table[NROW_PAD - 128:].reshape(16, 128)
    table_lin = _relayout(table.T, tail).reshape(NROW_PAD, D_MODEL)
    rows = _gather_rows(idx, table_lin, nchunk)
    rows = rows.reshape(B_pad, D_MODEL)[:B_total]
    return rows.reshape(x.shape[0], x.shape[1], D_MODEL)
